# baseline (speedup 1.0000x reference)
import functools

import jax
import jax.numpy as jnp
from jax import lax
from jax.experimental import pallas as pl
from jax.experimental.pallas import tpu as pltpu
from jax.experimental.pallas import tpu_sc as plsc

MAXL = 512
NH = 16
QLEN = 2048
KLEN = 2048
ETAB = 4104
NV = 11
VW = NV * 128
NCHUNK = VW // 16


@functools.partial(
    pl.kernel,
    out_type=jax.ShapeDtypeStruct((NH, QLEN, KLEN), jnp.float32),
    mesh=plsc.VectorSubcoreMesh(core_axis_name="c", subcore_axis_name="s"),
    scratch_types=[
        pltpu.VMEM((ETAB,), jnp.float32),
        pltpu.VMEM((8, VW), jnp.float32),
        pltpu.VMEM((8, VW), jnp.float32),
        pltpu.VMEM((8, 128), jnp.float32),
        pltpu.VMEM((8, 128), jnp.float32),
        pltpu.SemaphoreType.DMA,
        pltpu.SemaphoreType.DMA,
    ],
)
def _sc_bias(ef_hbm, out_hbm, etab, vb0, vb1, c0b, c1b, semv, semc):
    h = lax.axis_index("s")
    half = lax.axis_index("c")
    i0_half = half * 1024

    pltpu.sync_copy(ef_hbm.at[pl.ds(h * ETAB, ETAB)], etab)

    v0 = etab[pl.ds(0, 16)]
    v1 = etab[pl.ds(3584, 16)]
    for r in range(8):
        for k in range(8):
            c0b[r, pl.ds(16 * k, 16)] = v0
            c1b[r, pl.ds(16 * k, 16)] = v1

    vbufs = (vb0, vb1)

    def build(vb, x0v):
        for r in range(8):
            o = x0v - r
            for k in range(NCHUNK):
                vb[r, pl.ds(16 * k, 16)] = etab[pl.ds(o + 16 * k, 16)]

    def drain_varying():
        pltpu.make_async_copy(
            vb0, out_hbm.at[0, pl.ds(0, 8), pl.ds(0, VW)], semv
        ).wait()

    def drain_consts():
        for _ in range(5):
            pltpu.make_async_copy(
                c0b, out_hbm.at[0, pl.ds(0, 8), pl.ds(0, 128)], semc
            ).wait()

    def stripe(b, carry):
        i0 = i0_half + 8 * b
        x0 = 2047 - i0
        c_lo = jnp.clip((i0 - 888) // 128 + 1, 0, 16 - NV)
        col0 = pl.multiple_of(128 * c_lo, 128)
        i0a = pl.multiple_of(i0, 8)

        pl.when(b >= 2)(drain_varying)

        vb = vbufs[0]
        alt = vbufs[1]

        def do(vb_sel):
            build(vb_sel, x0 + col0)
            pltpu.async_copy(
                vb_sel, out_hbm.at[h, pl.ds(i0a, 8), pl.ds(col0, VW)], semv
            )

        pl.when(b % 2 == 0)(lambda: do(vb))
        pl.when(b % 2 == 1)(lambda: do(alt))

        def left(c, carry):
            colc = pl.multiple_of(128 * c, 128)
            pltpu.async_copy(
                c0b, out_hbm.at[h, pl.ds(i0a, 8), pl.ds(colc, 128)], semc
            )
            return carry

        def right(c, carry):
            colc = pl.multiple_of(128 * c, 128)
            pltpu.async_copy(
                c1b, out_hbm.at[h, pl.ds(i0a, 8), pl.ds(colc, 128)], semc
            )
            return carry

        lax.fori_loop(0, c_lo, left, 0)
        lax.fori_loop(c_lo + NV, 16, right, 0)
        pl.when(b >= 1)(drain_consts)
        return carry

    lax.fori_loop(0, 128, stripe, 0)
    for _ in range(2):
        drain_varying()
    drain_consts()


def kernel(qlen, klen, rel):
    del qlen, klen
    rel = rel.astype(jnp.float32)
    ef = jnp.concatenate(
        [
            jnp.broadcast_to(rel[0:1], (QLEN - MAXL - 1, NH)),
            rel,
            jnp.broadcast_to(rel[2 * MAXL : 2 * MAXL + 1], (ETAB - QLEN - MAXL, NH)),
        ],
        axis=0,
    ).T
    return _sc_bias(ef.reshape(-1))


# speedup vs baseline: 1.3044x; 1.3044x over previous
import functools

import jax
import jax.numpy as jnp
from jax import lax
from jax.experimental import pallas as pl
from jax.experimental.pallas import tpu as pltpu
from jax.experimental.pallas import tpu_sc as plsc

MAXL = 512
NH = 16
QLEN = 2048
KLEN = 2048
ETAB = 4104
NV = 11
VW = NV * 128
NCHUNK = VW // 16


@functools.partial(
    pl.kernel,
    out_type=jax.ShapeDtypeStruct((NH, QLEN, KLEN), jnp.float32),
    mesh=plsc.VectorSubcoreMesh(core_axis_name="c", subcore_axis_name="s"),
    scratch_types=[
        pltpu.VMEM((ETAB,), jnp.float32),
        pltpu.VMEM((8, VW), jnp.float32),
        pltpu.VMEM((8, VW), jnp.float32),
        pltpu.VMEM((8, 128), jnp.float32),
        pltpu.VMEM((8, 128), jnp.float32),
        pltpu.SemaphoreType.DMA,
        pltpu.SemaphoreType.DMA,
    ],
)
def _sc_bias(ef_hbm, out_hbm, etab, vb0, vb1, c0b, c1b, semv, semc):
    h = lax.axis_index("s")
    half = lax.axis_index("c")
    i0_half = half * 1024

    pltpu.sync_copy(ef_hbm.at[pl.ds(h * ETAB, ETAB)], etab)

    v0 = etab[pl.ds(0, 16)]
    v1 = etab[pl.ds(3584, 16)]
    for r in range(8):
        for k in range(8):
            c0b[r, pl.ds(16 * k, 16)] = v0
            c1b[r, pl.ds(16 * k, 16)] = v1

    vbufs = (vb0, vb1)

    def build(vb, x0v):
        def block(kk, carry):
            kbase = 176 * kk
            for r in range(8):
                o = x0v - r + kbase
                for j in range(11):
                    vb[r, pl.ds(kbase + 16 * j, 16)] = etab[pl.ds(o + 16 * j, 16)]
            return carry

        lax.fori_loop(0, 8, block, 0)

    def drain_varying():
        pltpu.make_async_copy(
            vb0, out_hbm.at[0, pl.ds(0, 8), pl.ds(0, VW)], semv
        ).wait()

    def drain_consts():
        for _ in range(5):
            pltpu.make_async_copy(
                c0b, out_hbm.at[0, pl.ds(0, 8), pl.ds(0, 128)], semc
            ).wait()

    def stripe(b, carry):
        i0 = i0_half + 8 * b
        x0 = 2047 - i0
        c_lo = jnp.clip((i0 - 888) // 128 + 1, 0, 16 - NV)
        col0 = pl.multiple_of(128 * c_lo, 128)
        i0a = pl.multiple_of(i0, 8)

        pl.when(b >= 2)(drain_varying)

        vb = vbufs[0]
        alt = vbufs[1]

        def do(vb_sel):
            build(vb_sel, x0 + col0)
            pltpu.async_copy(
                vb_sel, out_hbm.at[h, pl.ds(i0a, 8), pl.ds(col0, VW)], semv
            )

        pl.when(b % 2 == 0)(lambda: do(vb))
        pl.when(b % 2 == 1)(lambda: do(alt))

        def left(c, carry):
            colc = pl.multiple_of(128 * c, 128)
            pltpu.async_copy(
                c0b, out_hbm.at[h, pl.ds(i0a, 8), pl.ds(colc, 128)], semc
            )
            return carry

        def right(c, carry):
            colc = pl.multiple_of(128 * c, 128)
            pltpu.async_copy(
                c1b, out_hbm.at[h, pl.ds(i0a, 8), pl.ds(colc, 128)], semc
            )
            return carry

        lax.fori_loop(0, c_lo, left, 0)
        lax.fori_loop(c_lo + NV, 16, right, 0)
        pl.when(b >= 1)(drain_consts)
        return carry

    lax.fori_loop(0, 128, stripe, 0)
    for _ in range(2):
        drain_varying()
    drain_consts()


def kernel(qlen, klen, rel):
    del qlen, klen
    rel = rel.astype(jnp.float32)
    ef = jnp.concatenate(
        [
            jnp.broadcast_to(rel[0:1], (QLEN - MAXL - 1, NH)),
            rel,
            jnp.broadcast_to(rel[2 * MAXL : 2 * MAXL + 1], (ETAB - QLEN - MAXL, NH)),
        ],
        axis=0,
    ).T
    return _sc_bias(ef.reshape(-1))


# speedup vs baseline: 4.2371x; 3.2483x over previous
import functools

import jax
import jax.numpy as jnp
from jax import lax
from jax.experimental import pallas as pl
from jax.experimental.pallas import tpu as pltpu
from jax.experimental.pallas import tpu_sc as plsc

MAXL = 512
NH = 16
QLEN = 2048
KLEN = 2048
ETAB = 4104
NV = 11
VW = NV * 128
NCHUNK = VW // 16


@functools.partial(
    pl.kernel,
    out_type=jax.ShapeDtypeStruct((NH, QLEN, KLEN), jnp.float32),
    mesh=plsc.VectorSubcoreMesh(core_axis_name="c", subcore_axis_name="s"),
    scratch_types=[
        pltpu.VMEM((ETAB,), jnp.float32),
        pltpu.VMEM((8, VW), jnp.float32),
        pltpu.VMEM((8, VW), jnp.float32),
        pltpu.VMEM((8, 128), jnp.float32),
        pltpu.VMEM((8, 128), jnp.float32),
        pltpu.SemaphoreType.DMA,
        pltpu.SemaphoreType.DMA,
    ],
)
def _sc_bias(ef_hbm, out_hbm, etab, vb0, vb1, c0b, c1b, semv, semc):
    h = lax.axis_index("s")
    half = lax.axis_index("c")
    i0_half = half * 1024

    pltpu.sync_copy(ef_hbm.at[pl.ds(h * ETAB, ETAB)], etab)

    v0 = etab[pl.ds(0, 16)]
    v1 = etab[pl.ds(3584, 16)]
    for r in range(8):
        for k in range(8):
            c0b[r, pl.ds(16 * k, 16)] = v0
            c1b[r, pl.ds(16 * k, 16)] = v1

    vbufs = (vb0, vb1)

    def build(vb, x0v):
        @plsc.parallel_loop(0, NCHUNK, unroll=4)
        def block(k):
            kbase = 16 * k
            for r in range(8):
                vb[r, pl.ds(kbase, 16)] = etab[pl.ds(x0v - r + kbase, 16)]

    def drain_varying():
        pltpu.make_async_copy(
            vb0, out_hbm.at[0, pl.ds(0, 8), pl.ds(0, VW)], semv
        ).wait()

    def drain_consts():
        for _ in range(5):
            pltpu.make_async_copy(
                c0b, out_hbm.at[0, pl.ds(0, 8), pl.ds(0, 128)], semc
            ).wait()

    def stripe(b, carry):
        i0 = i0_half + 8 * b
        x0 = 2047 - i0
        c_lo = jnp.clip((i0 - 888) // 128 + 1, 0, 16 - NV)
        col0 = pl.multiple_of(128 * c_lo, 128)
        i0a = pl.multiple_of(i0, 8)

        pl.when(b >= 2)(drain_varying)

        vb = vbufs[0]
        alt = vbufs[1]

        def do(vb_sel):
            build(vb_sel, x0 + col0)
            pltpu.async_copy(
                vb_sel, out_hbm.at[h, pl.ds(i0a, 8), pl.ds(col0, VW)], semv
            )

        pl.when(b % 2 == 0)(lambda: do(vb))
        pl.when(b % 2 == 1)(lambda: do(alt))

        def left(c, carry):
            colc = pl.multiple_of(128 * c, 128)
            pltpu.async_copy(
                c0b, out_hbm.at[h, pl.ds(i0a, 8), pl.ds(colc, 128)], semc
            )
            return carry

        def right(c, carry):
            colc = pl.multiple_of(128 * c, 128)
            pltpu.async_copy(
                c1b, out_hbm.at[h, pl.ds(i0a, 8), pl.ds(colc, 128)], semc
            )
            return carry

        lax.fori_loop(0, c_lo, left, 0)
        lax.fori_loop(c_lo + NV, 16, right, 0)
        pl.when(b >= 1)(drain_consts)
        return carry

    lax.fori_loop(0, 128, stripe, 0)
    for _ in range(2):
        drain_varying()
    drain_consts()


def kernel(qlen, klen, rel):
    del qlen, klen
    rel = rel.astype(jnp.float32)
    ef = jnp.concatenate(
        [
            jnp.broadcast_to(rel[0:1], (QLEN - MAXL - 1, NH)),
            rel,
            jnp.broadcast_to(rel[2 * MAXL : 2 * MAXL + 1], (ETAB - QLEN - MAXL, NH)),
        ],
        axis=0,
    ).T
    return _sc_bias(ef.reshape(-1))
